# baseline (speedup 1.0000x reference)
import functools

import jax
import jax.numpy as jnp
from jax.experimental import pallas as pl
from jax.experimental.pallas import tpu as pltpu

N = 96
EPS = 1e-6
BLOCK_ROWS = 2048


def _dct2_matrix(n, dtype=jnp.float32):
    k = jnp.arange(n, dtype=dtype)[:, None]
    i = jnp.arange(n, dtype=dtype)[None, :]
    return 2.0 * jnp.cos(jnp.pi * (2.0 * i + 1.0) * k / (2.0 * n))


def _block_kernel(x_ref, a_ref, u1_ref, b1_ref, w2_ref, dbar_ref, g_ref,
                  bt_ref, o_ref):
    xb = x_ref[...]
    sx = jnp.sum(xb, axis=-1, keepdims=True)
    ssq = jnp.sum(xb * xb, axis=-1, keepdims=True)
    mu = jnp.sum(xb * dbar_ref[...], axis=-1, keepdims=True)
    mean_y2 = 2.0 * ssq + (2.0 / N) * sx * sx
    var = mean_y2 - mu * mu
    rs = jax.lax.rsqrt(var + EPS)

    t1 = jnp.dot(xb.astype(jnp.bfloat16), a_ref[...],
                 preferred_element_type=jnp.float32)
    h = jnp.maximum(rs * t1 - (rs * mu) * u1_ref[...] + b1_ref[...], 0.0)

    s_lin = jnp.dot(h.astype(jnp.bfloat16), w2_ref[...],
                    preferred_element_type=jnp.float32)
    s = jax.nn.sigmoid(s_lin)

    mu2 = jnp.mean(s, axis=-1, keepdims=True)
    d = s - mu2
    var2 = jnp.mean(d * d, axis=-1, keepdims=True)
    lw = d * jax.lax.rsqrt(var2 + EPS) * g_ref[...] + bt_ref[...]
    o_ref[...] = xb * lw


@jax.jit
def kernel(x, W1, W2, ln_gamma, ln_beta):
    b, c, l = x.shape
    m = b * c
    x2 = x.reshape(m, l)

    hp = jax.lax.Precision.HIGHEST
    D = _dct2_matrix(N, jnp.float32)
    W1t = W1.T
    A = jnp.dot(D.T, ln_gamma[:, None] * W1t, precision=hp)
    u1 = jnp.dot(ln_gamma[None, :], W1t, precision=hp)
    b1 = jnp.dot(ln_beta[None, :], W1t, precision=hp)
    dbar = jnp.sum(D, axis=0, keepdims=True) / N

    grid = (m // BLOCK_ROWS,)
    out = pl.pallas_call(
        _block_kernel,
        out_shape=jax.ShapeDtypeStruct((m, l), x.dtype),
        grid=grid,
        in_specs=[
            pl.BlockSpec((BLOCK_ROWS, l), lambda i: (i, 0)),
            pl.BlockSpec((N, 2 * N), lambda i: (0, 0)),
            pl.BlockSpec((1, 2 * N), lambda i: (0, 0)),
            pl.BlockSpec((1, 2 * N), lambda i: (0, 0)),
            pl.BlockSpec((2 * N, N), lambda i: (0, 0)),
            pl.BlockSpec((1, N), lambda i: (0, 0)),
            pl.BlockSpec((1, N), lambda i: (0, 0)),
            pl.BlockSpec((1, N), lambda i: (0, 0)),
        ],
        out_specs=pl.BlockSpec((BLOCK_ROWS, l), lambda i: (i, 0)),
        compiler_params=pltpu.CompilerParams(
            dimension_semantics=("parallel",),
            vmem_limit_bytes=56 * 1024 * 1024,
        ),
        name="dct_channel_block",
    )(
        x2,
        A.astype(jnp.bfloat16),
        u1,
        b1,
        W2.T.astype(jnp.bfloat16),
        dbar,
        ln_gamma[None, :],
        ln_beta[None, :],
    )
    return out.reshape(b, c, l)
